# Initial kernel scaffold; baseline (speedup 1.0000x reference)
import functools

import jax
import jax.numpy as jnp
from jax import lax
from jax.experimental import pallas as pl
from jax.experimental.pallas import tpu as pltpu
from jax.experimental.pallas import tpu_sc as plsc

D = 32
_INV_SQRT_D = 1.0 / (32.0 ** 0.5)


_T = 256


def _attn_body(h_ref, wqt_ref, wkt_ref, a_ref):
    hb = h_ref[...]
    q = jnp.dot(hb, wqt_ref[...], preferred_element_type=jnp.float32)
    k = jnp.dot(hb, wkt_ref[...], preferred_element_type=jnp.float32)
    e3 = jnp.exp(q[:, :, None] * (k[:, None, :] * _INV_SQRT_D))
    denom = jnp.sum(e3, axis=1)
    g = hb / denom
    a_ref[...] = jnp.sum(e3 * g[:, None, :], axis=2)


def _node_attention(h_pad, wqt, wkt):
    n_pad = h_pad.shape[0]
    grid = n_pad // _T
    return pl.pallas_call(
        _attn_body,
        grid=(grid,),
        in_specs=[
            pl.BlockSpec((_T, D), lambda i: (i, 0)),
            pl.BlockSpec((D, D), lambda i: (0, 0)),
            pl.BlockSpec((D, D), lambda i: (0, 0)),
        ],
        out_specs=pl.BlockSpec((_T, D), lambda i: (i, 0)),
        out_shape=jax.ShapeDtypeStruct((n_pad, D), jnp.float32),
        compiler_params=pltpu.CompilerParams(
            dimension_semantics=("arbitrary",)),
    )(h_pad, wqt, wkt)


_CH = 128
_NW = 32
_NSUB = 16
_CW = 16


def _edge_kernel(n_acc, n_chunks):
    rpt = n_acc // _NSUB
    mesh = plsc.VectorSubcoreMesh(core_axis_name="c", subcore_axis_name="s")

    @functools.partial(
        pl.kernel,
        out_type=[
            jax.ShapeDtypeStruct((2, n_acc, D), jnp.float32),
            jax.ShapeDtypeStruct((2, n_acc, _CW), jnp.float32),
        ],
        mesh=mesh,
        scratch_types=[
            pltpu.VMEM((n_chunks, _CH), jnp.int32),
            pltpu.VMEM((n_chunks, _CH), jnp.int32),
            pltpu.VMEM((_CH, D), jnp.float32),
            pltpu.VMEM((_CH, _CW), jnp.float32),
            pltpu.VMEM_SHARED((n_acc, D), jnp.float32),
            pltpu.VMEM_SHARED((n_acc, _CW), jnp.float32),
            pltpu.SemaphoreType.DMA,
        ],
    )
    def edge(a_hbm, src_hbm, dst_hbm, zsum_hbm, zcnt_hbm, ones_hbm,
             psum_hbm, pcnt_hbm,
             src_v, dst_v, rows_v, ones_v, ssum, scnt, gsem):
        cid = lax.axis_index("c")
        sid = lax.axis_index("s")
        wid = cid * _NSUB + sid
        row0 = sid * rpt
        pltpu.sync_copy(zsum_hbm.at[pl.ds(row0, rpt)], ssum.at[pl.ds(row0, rpt)])
        pltpu.sync_copy(zcnt_hbm.at[pl.ds(row0, rpt)], scnt.at[pl.ds(row0, rpt)])
        pltpu.sync_copy(ones_hbm, ones_v)
        pltpu.sync_copy(src_hbm.at[pl.ds(wid * n_chunks, n_chunks)], src_v)
        pltpu.sync_copy(dst_hbm.at[pl.ds(wid * n_chunks, n_chunks)], dst_v)
        plsc.subcore_barrier()

        def chunk(c, carry):
            pltpu.async_copy(a_hbm.at[src_v.at[c]], rows_v, gsem).wait()
            pltpu.sync_copy(rows_v, ssum.at[dst_v.at[c]], add=True)
            pltpu.sync_copy(ones_v, scnt.at[dst_v.at[c]], add=True)
            return carry

        lax.fori_loop(0, n_chunks, chunk, 0)
        plsc.subcore_barrier()
        pltpu.sync_copy(ssum.at[pl.ds(row0, rpt)],
                        psum_hbm.at[cid, pl.ds(row0, rpt)])
        pltpu.sync_copy(scnt.at[pl.ds(row0, rpt)],
                        pcnt_hbm.at[cid, pl.ds(row0, rpt)])

    return edge


def _combine_body(ps_ref, pc_ref, o_ref):
    s = ps_ref[0] + ps_ref[1]
    c = pc_ref[0, :, :1] + pc_ref[1, :, :1]
    o_ref[...] = s / jnp.maximum(c, 1.0)


def _combine(psum, pcnt, n):
    tile = 1000
    return pl.pallas_call(
        _combine_body,
        grid=(n // tile,),
        in_specs=[
            pl.BlockSpec((2, tile, D), lambda i: (0, i, 0)),
            pl.BlockSpec((2, tile, _CW), lambda i: (0, i, 0)),
        ],
        out_specs=pl.BlockSpec((tile, D), lambda i: (i, 0)),
        out_shape=jax.ShapeDtypeStruct((n, D), jnp.float32),
        compiler_params=pltpu.CompilerParams(
            dimension_semantics=("arbitrary",)),
    )(psum, pcnt)


def kernel(h, edge_index, Wq, Wk):
    n, d = h.shape
    e = edge_index.shape[1]

    n_pad = ((n + _T - 1) // _T) * _T
    h_pad = jnp.pad(h, ((0, n_pad - n), (0, 0)))
    a = _node_attention(h_pad, Wq.T, Wk.T)

    n_acc = ((n + 1 + _NSUB - 1) // _NSUB) * _NSUB
    epw = ((e + _NW * _CH - 1) // (_NW * _CH)) * _CH
    n_chunks = epw // _CH
    e_pad = _NW * epw
    src = jnp.concatenate(
        [edge_index[0], jnp.zeros((e_pad - e,), jnp.int32)]).reshape(-1, _CH)
    dst = jnp.concatenate(
        [edge_index[1], jnp.full((e_pad - e,), n, jnp.int32)]).reshape(-1, _CH)

    zsum = jnp.zeros((n_acc, D), jnp.float32)
    zcnt = jnp.zeros((n_acc, _CW), jnp.float32)
    ones = jnp.ones((_CH, _CW), jnp.float32)

    psum, pcnt = _edge_kernel(n_acc, n_chunks)(a, src, dst, zsum, zcnt, ones)
    return _combine(psum, pcnt, n)


# speedup vs baseline: 8.4090x; 8.4090x over previous
import functools

import jax
import jax.numpy as jnp
from jax import lax
from jax.experimental import pallas as pl
from jax.experimental.pallas import tpu as pltpu
from jax.experimental.pallas import tpu_sc as plsc

D = 32
_INV_SQRT_D = 1.0 / (32.0 ** 0.5)


_T = 256


def _attn_body(h_ref, wqt_ref, wkt_ref, a_ref):
    hb = h_ref[...]
    q = jnp.dot(hb, wqt_ref[...], preferred_element_type=jnp.float32)
    k = jnp.dot(hb, wkt_ref[...], preferred_element_type=jnp.float32)
    e3 = jnp.exp(q[:, :, None] * (k[:, None, :] * _INV_SQRT_D))
    denom = jnp.sum(e3, axis=1)
    g = hb / denom
    a_ref[...] = jnp.sum(e3 * g[:, None, :], axis=2)


def _node_attention(h_pad, wqt, wkt):
    n_pad = h_pad.shape[0]
    grid = n_pad // _T
    return pl.pallas_call(
        _attn_body,
        grid=(grid,),
        in_specs=[
            pl.BlockSpec((_T, D), lambda i: (i, 0)),
            pl.BlockSpec((D, D), lambda i: (0, 0)),
            pl.BlockSpec((D, D), lambda i: (0, 0)),
        ],
        out_specs=pl.BlockSpec((_T, D), lambda i: (i, 0)),
        out_shape=jax.ShapeDtypeStruct((n_pad, D), jnp.float32),
        compiler_params=pltpu.CompilerParams(
            dimension_semantics=("arbitrary",)),
    )(h_pad, wqt, wkt)


_CH = 128
_NW = 32
_NSUB = 16
_CW = 16


def _edge_kernel(n_acc, n_chunks, row_stride):
    rpt = n_acc // _NSUB
    mesh = plsc.VectorSubcoreMesh(core_axis_name="c", subcore_axis_name="s")

    @functools.partial(
        pl.kernel,
        out_type=[
            jax.ShapeDtypeStruct((2, n_acc, D), jnp.float32),
            jax.ShapeDtypeStruct((2, n_acc, _CW), jnp.float32),
        ],
        mesh=mesh,
        scratch_types=[
            pltpu.VMEM((row_stride, _CH), jnp.int32),
            pltpu.VMEM((row_stride, _CH), jnp.int32),
            pltpu.VMEM((_CH, D), jnp.float32),
            pltpu.VMEM((_CH, _CW), jnp.float32),
            pltpu.VMEM_SHARED((n_acc, D), jnp.float32),
            pltpu.VMEM_SHARED((n_acc, _CW), jnp.float32),
            pltpu.SemaphoreType.DMA,
        ],
        compiler_params=pltpu.CompilerParams(use_tc_tiling_on_sc=False),
    )
    def edge(a_hbm, src_hbm, dst_hbm, zsum_hbm, zcnt_hbm, ones_hbm,
             psum_hbm, pcnt_hbm,
             src_v, dst_v, rows_v, ones_v, ssum, scnt, gsem):
        cid = lax.axis_index("c")
        sid = lax.axis_index("s")
        wid = cid * _NSUB + sid
        row0 = sid * rpt
        pltpu.sync_copy(zsum_hbm.at[pl.ds(row0, rpt)], ssum.at[pl.ds(row0, rpt)])
        pltpu.sync_copy(zcnt_hbm.at[pl.ds(row0, rpt)], scnt.at[pl.ds(row0, rpt)])
        pltpu.sync_copy(ones_hbm, ones_v)
        pltpu.sync_copy(src_hbm.at[pl.ds(wid * row_stride, row_stride)], src_v)
        pltpu.sync_copy(dst_hbm.at[pl.ds(wid * row_stride, row_stride)], dst_v)
        plsc.subcore_barrier()

        def chunk(c, carry):
            pltpu.async_copy(a_hbm.at[src_v.at[c]], rows_v, gsem).wait()
            pltpu.sync_copy(rows_v, ssum.at[dst_v.at[c]], add=True)
            pltpu.sync_copy(ones_v, scnt.at[dst_v.at[c]], add=True)
            return carry

        lax.fori_loop(0, n_chunks, chunk, 0)
        plsc.subcore_barrier()
        pltpu.sync_copy(ssum.at[pl.ds(row0, rpt)],
                        psum_hbm.at[cid, pl.ds(row0, rpt)])
        pltpu.sync_copy(scnt.at[pl.ds(row0, rpt)],
                        pcnt_hbm.at[cid, pl.ds(row0, rpt)])

    return edge


def _combine_body(ps_ref, pc_ref, o_ref):
    s = ps_ref[0] + ps_ref[1]
    c = pc_ref[0, :, :1] + pc_ref[1, :, :1]
    o_ref[...] = s / jnp.maximum(c, 1.0)


def _combine(psum, pcnt, n):
    tile = 1000
    return pl.pallas_call(
        _combine_body,
        grid=(n // tile,),
        in_specs=[
            pl.BlockSpec((2, tile, D), lambda i: (0, i, 0)),
            pl.BlockSpec((2, tile, _CW), lambda i: (0, i, 0)),
        ],
        out_specs=pl.BlockSpec((tile, D), lambda i: (i, 0)),
        out_shape=jax.ShapeDtypeStruct((n, D), jnp.float32),
        compiler_params=pltpu.CompilerParams(
            dimension_semantics=("arbitrary",)),
    )(psum, pcnt)


def kernel(h, edge_index, Wq, Wk):
    n, d = h.shape
    e = edge_index.shape[1]

    n_pad = ((n + _T - 1) // _T) * _T
    h_pad = jnp.pad(h, ((0, n_pad - n), (0, 0)))
    a = _node_attention(h_pad, Wq.T, Wk.T)

    n_acc = ((n + 1 + 127) // 128) * 128
    epw = ((e + _NW * _CH - 1) // (_NW * _CH)) * _CH
    n_chunks = epw // _CH
    row_stride = ((n_chunks + 7) // 8) * 8
    e_pad = _NW * epw

    def _layout(idx, fill):
        x = jnp.concatenate([idx, jnp.full((e_pad - e,), fill, jnp.int32)])
        x = x.reshape(_NW, n_chunks, _CH)
        x = jnp.pad(x, ((0, 0), (0, row_stride - n_chunks), (0, 0)))
        return x.reshape(_NW * row_stride, _CH)

    src = _layout(edge_index[0], 0)
    dst = _layout(edge_index[1], n)

    zsum = jnp.zeros((n_acc, D), jnp.float32)
    zcnt = jnp.zeros((n_acc, _CW), jnp.float32)
    ones = jnp.ones((_CH, _CW), jnp.float32)

    psum, pcnt = _edge_kernel(n_acc, n_chunks, row_stride)(
        a, src, dst, zsum, zcnt, ones)
    return _combine(psum, pcnt, n)


# speedup vs baseline: 12.7710x; 1.5187x over previous
import functools

import jax
import jax.numpy as jnp
from jax import lax
from jax.experimental import pallas as pl
from jax.experimental.pallas import tpu as pltpu
from jax.experimental.pallas import tpu_sc as plsc

D = 32
_INV_SQRT_D = 1.0 / (32.0 ** 0.5)


_T = 256


def _attn_body(h_ref, wq_ref, wk_ref, a_ref):
    hb = h_ref[...]
    hT = hb.T
    qT = jnp.dot(wq_ref[...], hT, preferred_element_type=jnp.float32)
    kT = jnp.dot(wk_ref[...], hT,
                 preferred_element_type=jnp.float32) * _INV_SQRT_D
    e3 = jnp.exp(qT[:, None, :] * kT[None, :, :])
    denom = jnp.sum(e3, axis=0)
    gT = hT / denom
    aT = jnp.sum(e3 * gT[None, :, :], axis=1)
    a_ref[...] = aT.T


def _node_attention(h_pad, wq, wk):
    n_pad = h_pad.shape[0]
    grid = n_pad // _T
    return pl.pallas_call(
        _attn_body,
        grid=(grid,),
        in_specs=[
            pl.BlockSpec((_T, D), lambda i: (i, 0)),
            pl.BlockSpec((D, D), lambda i: (0, 0)),
            pl.BlockSpec((D, D), lambda i: (0, 0)),
        ],
        out_specs=pl.BlockSpec((_T, D), lambda i: (i, 0)),
        out_shape=jax.ShapeDtypeStruct((n_pad, D), jnp.float32),
        compiler_params=pltpu.CompilerParams(
            dimension_semantics=("arbitrary",)),
    )(h_pad, wq, wk)


_CH = 128
_NW = 32
_NSUB = 16
_CW = 16


def _edge_kernel(n_acc, n_chunks, row_stride):
    rpt = n_acc // _NSUB
    mesh = plsc.VectorSubcoreMesh(core_axis_name="c", subcore_axis_name="s")

    @functools.partial(
        pl.kernel,
        out_type=[
            jax.ShapeDtypeStruct((2, n_acc, D), jnp.float32),
            jax.ShapeDtypeStruct((2, n_acc, _CW), jnp.float32),
        ],
        mesh=mesh,
        scratch_types=[
            pltpu.VMEM((row_stride, _CH), jnp.int32),
            pltpu.VMEM((row_stride, _CH), jnp.int32),
            pltpu.VMEM((_CH, D), jnp.float32),
            pltpu.VMEM((_CH, _CW), jnp.float32),
            pltpu.VMEM_SHARED((n_acc, D), jnp.float32),
            pltpu.VMEM_SHARED((n_acc, _CW), jnp.float32),
            pltpu.SemaphoreType.DMA,
        ],
        compiler_params=pltpu.CompilerParams(use_tc_tiling_on_sc=False),
    )
    def edge(a_hbm, src_hbm, dst_hbm, zsum_hbm, zcnt_hbm, ones_hbm,
             psum_hbm, pcnt_hbm,
             src_v, dst_v, rows_v, ones_v, ssum, scnt, gsem):
        cid = lax.axis_index("c")
        sid = lax.axis_index("s")
        wid = cid * _NSUB + sid
        row0 = sid * rpt
        pltpu.sync_copy(zsum_hbm.at[pl.ds(row0, rpt)], ssum.at[pl.ds(row0, rpt)])
        pltpu.sync_copy(zcnt_hbm.at[pl.ds(row0, rpt)], scnt.at[pl.ds(row0, rpt)])
        pltpu.sync_copy(ones_hbm, ones_v)
        pltpu.sync_copy(src_hbm.at[pl.ds(wid * row_stride, row_stride)], src_v)
        pltpu.sync_copy(dst_hbm.at[pl.ds(wid * row_stride, row_stride)], dst_v)
        plsc.subcore_barrier()

        def chunk(c, carry):
            pltpu.async_copy(a_hbm.at[src_v.at[c]], rows_v, gsem).wait()
            pltpu.sync_copy(rows_v, ssum.at[dst_v.at[c]], add=True)
            pltpu.sync_copy(ones_v, scnt.at[dst_v.at[c]], add=True)
            return carry

        lax.fori_loop(0, n_chunks, chunk, 0)
        plsc.subcore_barrier()
        pltpu.sync_copy(ssum.at[pl.ds(row0, rpt)],
                        psum_hbm.at[cid, pl.ds(row0, rpt)])
        pltpu.sync_copy(scnt.at[pl.ds(row0, rpt)],
                        pcnt_hbm.at[cid, pl.ds(row0, rpt)])

    return edge


def _combine_body(ps_ref, pc_ref, o_ref):
    s = ps_ref[0] + ps_ref[1]
    c = pc_ref[0, :, :1] + pc_ref[1, :, :1]
    o_ref[...] = s / jnp.maximum(c, 1.0)


def _combine(psum, pcnt, n):
    tile = 1000
    return pl.pallas_call(
        _combine_body,
        grid=(n // tile,),
        in_specs=[
            pl.BlockSpec((2, tile, D), lambda i: (0, i, 0)),
            pl.BlockSpec((2, tile, _CW), lambda i: (0, i, 0)),
        ],
        out_specs=pl.BlockSpec((tile, D), lambda i: (i, 0)),
        out_shape=jax.ShapeDtypeStruct((n, D), jnp.float32),
        compiler_params=pltpu.CompilerParams(
            dimension_semantics=("arbitrary",)),
    )(psum, pcnt)


def kernel(h, edge_index, Wq, Wk):
    n, d = h.shape
    e = edge_index.shape[1]

    n_pad = ((n + _T - 1) // _T) * _T
    h_pad = jnp.pad(h, ((0, n_pad - n), (0, 0)))
    a = _node_attention(h_pad, Wq, Wk)

    n_acc = ((n + 1 + 127) // 128) * 128
    epw = ((e + _NW * _CH - 1) // (_NW * _CH)) * _CH
    n_chunks = epw // _CH
    row_stride = ((n_chunks + 7) // 8) * 8
    e_pad = _NW * epw

    def _layout(idx, fill):
        x = jnp.concatenate([idx, jnp.full((e_pad - e,), fill, jnp.int32)])
        x = x.reshape(_NW, n_chunks, _CH)
        x = jnp.pad(x, ((0, 0), (0, row_stride - n_chunks), (0, 0)))
        return x.reshape(_NW * row_stride, _CH)

    src = _layout(edge_index[0], 0)
    dst = _layout(edge_index[1], n)

    zsum = jnp.zeros((n_acc, D), jnp.float32)
    zcnt = jnp.zeros((n_acc, _CW), jnp.float32)
    ones = jnp.ones((_CH, _CW), jnp.float32)

    psum, pcnt = _edge_kernel(n_acc, n_chunks, row_stride)(
        a, src, dst, zsum, zcnt, ones)
    return _combine(psum, pcnt, n)


# speedup vs baseline: 17.4553x; 1.3668x over previous
import functools

import jax
import jax.numpy as jnp
from jax import lax
from jax.experimental import pallas as pl
from jax.experimental.pallas import tpu as pltpu
from jax.experimental.pallas import tpu_sc as plsc

D = 32
_INV_SQRT_D = 1.0 / (32.0 ** 0.5)


_T = 512


def _attn_body(h_ref, wq_ref, wk_ref, a_ref):
    hb = h_ref[...]
    hT = hb.T
    qT = jnp.dot(wq_ref[...], hT, preferred_element_type=jnp.float32)
    kT = jnp.dot(wk_ref[...], hT,
                 preferred_element_type=jnp.float32) * _INV_SQRT_D
    e3 = jnp.exp(qT[:, None, :] * kT[None, :, :])
    denom = jnp.sum(e3, axis=0)
    gT = hT / denom
    aT = jnp.sum(e3 * gT[None, :, :], axis=1)
    a_ref[...] = aT.T


def _node_attention(h_pad, wq, wk):
    n_pad = h_pad.shape[0]
    grid = n_pad // _T
    return pl.pallas_call(
        _attn_body,
        grid=(grid,),
        in_specs=[
            pl.BlockSpec((_T, D), lambda i: (i, 0)),
            pl.BlockSpec((D, D), lambda i: (0, 0)),
            pl.BlockSpec((D, D), lambda i: (0, 0)),
        ],
        out_specs=pl.BlockSpec((_T, D), lambda i: (i, 0)),
        out_shape=jax.ShapeDtypeStruct((n_pad, D), jnp.float32),
        compiler_params=pltpu.CompilerParams(
            dimension_semantics=("arbitrary",)),
    )(h_pad, wq, wk)


_CH = 128
_NW = 32
_NSUB = 16
_CW = 16


def _edge_kernel(n_acc, n_chunks):
    rpt = n_acc // _NSUB
    epw = n_chunks * _CH
    mesh = plsc.VectorSubcoreMesh(core_axis_name="c", subcore_axis_name="s")

    @functools.partial(
        pl.kernel,
        out_type=[
            jax.ShapeDtypeStruct((2, n_acc, D), jnp.float32),
            jax.ShapeDtypeStruct((2, n_acc, _CW), jnp.float32),
        ],
        mesh=mesh,
        scratch_types=[
            pltpu.VMEM((epw,), jnp.int32),
            pltpu.VMEM((epw,), jnp.int32),
            pltpu.VMEM((_CH, D), jnp.float32),
            pltpu.VMEM((_CH, D), jnp.float32),
            pltpu.VMEM((_CH, _CW), jnp.float32),
            pltpu.VMEM_SHARED((n_acc, D), jnp.float32),
            pltpu.VMEM_SHARED((n_acc, _CW), jnp.float32),
            pltpu.SemaphoreType.DMA,
            pltpu.SemaphoreType.DMA,
            pltpu.SemaphoreType.DMA,
            pltpu.SemaphoreType.DMA,
            pltpu.SemaphoreType.DMA,
            pltpu.SemaphoreType.DMA,
        ],
        compiler_params=pltpu.CompilerParams(use_tc_tiling_on_sc=False),
    )
    def edge(a_hbm, src_hbm, dst_hbm, zsum_hbm, zcnt_hbm, ones_hbm,
             psum_hbm, pcnt_hbm,
             src_v, dst_v, rows0, rows1, ones_v, ssum, scnt,
             g0, g1, s0, s1, c0, c1):
        cid = lax.axis_index("c")
        sid = lax.axis_index("s")
        wid = cid * _NSUB + sid
        row0 = sid * rpt
        pltpu.sync_copy(zsum_hbm.at[pl.ds(row0, rpt)], ssum.at[pl.ds(row0, rpt)])
        pltpu.sync_copy(zcnt_hbm.at[pl.ds(row0, rpt)], scnt.at[pl.ds(row0, rpt)])
        pltpu.sync_copy(ones_hbm, ones_v)
        pltpu.sync_copy(src_hbm.at[pl.ds(wid * epw, epw)], src_v)
        pltpu.sync_copy(dst_hbm.at[pl.ds(wid * epw, epw)], dst_v)
        plsc.subcore_barrier()

        rows = (rows0, rows1)
        gsem = (g0, g1)
        ssem = (s0, s1)
        csem = (c0, c1)
        gathers = [None] * n_chunks
        sums = [None] * n_chunks
        cnts = [None] * n_chunks
        gathers[0] = pltpu.async_copy(
            a_hbm.at[src_v.at[pl.ds(0, _CH)]], rows[0], gsem[0])
        for c in range(n_chunks):
            b = c & 1
            if c >= 1:
                sums[c - 1].wait()
                cnts[c - 1].wait()
            if c + 1 < n_chunks:
                gathers[c + 1] = pltpu.async_copy(
                    a_hbm.at[src_v.at[pl.ds((c + 1) * _CH, _CH)]],
                    rows[1 - b], gsem[1 - b])
            gathers[c].wait()
            dix = dst_v.at[pl.ds(c * _CH, _CH)]
            sums[c] = pltpu.async_copy(rows[b], ssum.at[dix], ssem[b], add=True)
            cnts[c] = pltpu.async_copy(ones_v, scnt.at[dix], csem[b], add=True)
        sums[n_chunks - 1].wait()
        cnts[n_chunks - 1].wait()
        plsc.subcore_barrier()
        pltpu.sync_copy(ssum.at[pl.ds(row0, rpt)],
                        psum_hbm.at[cid, pl.ds(row0, rpt)])
        pltpu.sync_copy(scnt.at[pl.ds(row0, rpt)],
                        pcnt_hbm.at[cid, pl.ds(row0, rpt)])

    return edge


def _combine_kernel(n_acc, n_out):
    base = (n_out // _NW) // 8 * 8
    tail0 = _NW * base
    tail = n_out - tail0
    mesh = plsc.VectorSubcoreMesh(core_axis_name="c", subcore_axis_name="s")

    @functools.partial(
        pl.kernel,
        out_type=jax.ShapeDtypeStruct((n_out, D), jnp.float32),
        mesh=mesh,
        scratch_types=[
            pltpu.VMEM((base + tail, D), jnp.float32),
            pltpu.VMEM((base + tail, D), jnp.float32),
            pltpu.VMEM((base + tail, _CW), jnp.float32),
            pltpu.VMEM((base + tail, _CW), jnp.float32),
            pltpu.VMEM((base + tail, D), jnp.float32),
        ],
        compiler_params=pltpu.CompilerParams(use_tc_tiling_on_sc=False),
    )
    def combine(psum_hbm, pcnt_hbm, out_hbm, p0, p1, c0, c1, o):
        cid = lax.axis_index("c")
        sid = lax.axis_index("s")
        wid = cid * _NSUB + sid
        nr = jnp.where(wid == 0, base + tail, base)
        row0 = jnp.where(wid == 0, 0, wid * base + tail)

        def do(nrows):
            pltpu.sync_copy(psum_hbm.at[0, pl.ds(row0, nrows)],
                            p0.at[pl.ds(0, nrows)])
            pltpu.sync_copy(psum_hbm.at[1, pl.ds(row0, nrows)],
                            p1.at[pl.ds(0, nrows)])
            pltpu.sync_copy(pcnt_hbm.at[0, pl.ds(row0, nrows)],
                            c0.at[pl.ds(0, nrows)])
            pltpu.sync_copy(pcnt_hbm.at[1, pl.ds(row0, nrows)],
                            c1.at[pl.ds(0, nrows)])

            def body(r, carry):
                cnt = jnp.maximum(c0[r, pl.ds(0, 16)] + c1[r, pl.ds(0, 16)],
                                  1.0)
                o[r, pl.ds(0, 16)] = (p0[r, pl.ds(0, 16)]
                                      + p1[r, pl.ds(0, 16)]) / cnt
                o[r, pl.ds(16, 16)] = (p0[r, pl.ds(16, 16)]
                                       + p1[r, pl.ds(16, 16)]) / cnt
                return carry

            lax.fori_loop(0, nrows, body, 0)
            pltpu.sync_copy(o.at[pl.ds(0, nrows)],
                            out_hbm.at[pl.ds(row0, nrows)])

        @pl.when(wid == 0)
        def _():
            do(base + tail)

        @pl.when(wid != 0)
        def _():
            do(base)

    return combine


def kernel(h, edge_index, Wq, Wk):
    n, d = h.shape
    e = edge_index.shape[1]

    n_pad = ((n + _T - 1) // _T) * _T
    h_pad = jnp.pad(h, ((0, n_pad - n), (0, 0)))
    a = _node_attention(h_pad, Wq, Wk)

    n_acc = ((n + 1 + 127) // 128) * 128
    epw = ((e + _NW * _CH - 1) // (_NW * _CH)) * _CH
    n_chunks = epw // _CH
    e_pad = _NW * epw
    ei = jnp.pad(edge_index, ((0, 0), (0, e_pad - e)), constant_values=n)
    src, dst = ei[0], ei[1]

    zsum = jnp.zeros((n_acc, D), jnp.float32)
    zcnt = jnp.zeros((n_acc, _CW), jnp.float32)
    ones = jnp.ones((_CH, _CW), jnp.float32)

    psum, pcnt = _edge_kernel(n_acc, n_chunks)(a, src, dst, zsum, zcnt, ones)
    return _combine_kernel(n_acc, n)(psum, pcnt)


# speedup vs baseline: 17.7579x; 1.0173x over previous
import functools

import jax
import jax.numpy as jnp
from jax import lax
from jax.experimental import pallas as pl
from jax.experimental.pallas import tpu as pltpu
from jax.experimental.pallas import tpu_sc as plsc

D = 32
_INV_SQRT_D = 1.0 / (32.0 ** 0.5)


_T = 512


def _attn_body(h_ref, wq_ref, wk_ref, a_ref):
    hb = h_ref[...]
    hT = hb.T
    qT = jnp.dot(wq_ref[...], hT, preferred_element_type=jnp.float32)
    kT = jnp.dot(wk_ref[...], hT,
                 preferred_element_type=jnp.float32) * _INV_SQRT_D
    e3 = jnp.exp(qT[:, None, :] * kT[None, :, :])
    denom = jnp.sum(e3, axis=0)
    gT = hT / denom
    aT = jnp.sum(e3 * gT[None, :, :], axis=1)
    a_ref[...] = aT.T


def _node_attention(h_pad, wq, wk):
    grid = (h_pad.shape[0] + _T - 1) // _T
    n_pad = grid * _T
    return pl.pallas_call(
        _attn_body,
        grid=(grid,),
        in_specs=[
            pl.BlockSpec((_T, D), lambda i: (i, 0)),
            pl.BlockSpec((D, D), lambda i: (0, 0)),
            pl.BlockSpec((D, D), lambda i: (0, 0)),
        ],
        out_specs=pl.BlockSpec((_T, D), lambda i: (i, 0)),
        out_shape=jax.ShapeDtypeStruct((n_pad, D), jnp.float32),
        compiler_params=pltpu.CompilerParams(
            dimension_semantics=("arbitrary",)),
    )(h_pad, wq, wk)


_CH = 128
_NW = 32
_NSUB = 16
_CW = 16


def _edge_kernel(n_acc, n_chunks):
    rpt = n_acc // _NSUB
    epw = n_chunks * _CH
    n_pairs = n_chunks // 2
    mesh = plsc.VectorSubcoreMesh(core_axis_name="c", subcore_axis_name="s")

    @functools.partial(
        pl.kernel,
        out_type=[
            jax.ShapeDtypeStruct((2, n_acc, D), jnp.float32),
            jax.ShapeDtypeStruct((2, n_acc, _CW), jnp.float32),
        ],
        mesh=mesh,
        scratch_types=[
            pltpu.VMEM((epw,), jnp.int32),
            pltpu.VMEM((epw,), jnp.int32),
            pltpu.VMEM((_CH, D), jnp.float32),
            pltpu.VMEM((_CH, D), jnp.float32),
            pltpu.VMEM((_CH, _CW), jnp.float32),
            pltpu.VMEM_SHARED((n_acc, D), jnp.float32),
            pltpu.VMEM_SHARED((n_acc, _CW), jnp.float32),
            pltpu.SemaphoreType.DMA,
            pltpu.SemaphoreType.DMA,
            pltpu.SemaphoreType.DMA,
            pltpu.SemaphoreType.DMA,
            pltpu.SemaphoreType.DMA,
            pltpu.SemaphoreType.DMA,
        ],
        compiler_params=pltpu.CompilerParams(use_tc_tiling_on_sc=False),
    )
    def edge(a_hbm, ei_hbm, zsum_hbm, zcnt_hbm, ones_hbm,
             psum_hbm, pcnt_hbm,
             src_v, dst_v, rows0, rows1, ones_v, ssum, scnt,
             ga, gb, sa, sb, ca, cb):
        cid = lax.axis_index("c")
        sid = lax.axis_index("s")
        wid = cid * _NSUB + sid
        row0 = sid * rpt
        pltpu.sync_copy(zsum_hbm.at[pl.ds(row0, rpt)], ssum.at[pl.ds(row0, rpt)])
        pltpu.sync_copy(zcnt_hbm.at[pl.ds(row0, rpt)], scnt.at[pl.ds(row0, rpt)])
        pltpu.sync_copy(ones_hbm, ones_v)
        pltpu.sync_copy(ei_hbm.at[0, pl.ds(wid * epw, epw)], src_v)
        pltpu.sync_copy(ei_hbm.at[1, pl.ds(wid * epw, epw)], dst_v)
        plsc.subcore_barrier()

        def _gather(c, buf, sem):
            pltpu.async_copy(a_hbm.at[src_v.at[pl.ds(c * _CH, _CH)]], buf, sem)

        def _scatter(c, buf, sem, csem_):
            dix = dst_v.at[pl.ds(c * _CH, _CH)]
            pltpu.async_copy(buf, ssum.at[dix], sem, add=True)
            pltpu.async_copy(ones_v, scnt.at[dix], csem_, add=True)

        def _wait(buf, sem):
            if buf is ones_v:
                pltpu.make_async_copy(ones_hbm, buf, sem).wait()
            else:
                pltpu.make_async_copy(a_hbm.at[src_v.at[pl.ds(0, _CH)]],
                                      buf, sem).wait()

        _gather(0, rows0, ga)

        def pair(i, carry):
            c0 = 2 * i
            _gather(c0 + 1, rows1, gb)
            _wait(rows0, ga)
            _scatter(c0, rows0, sa, ca)
            _wait(rows1, gb)
            _scatter(c0 + 1, rows1, sb, cb)
            _wait(rows0, sa)
            _wait(ones_v, ca)

            @pl.when(i + 1 < n_pairs)
            def _():
                _gather(c0 + 2, rows0, ga)

            _wait(rows1, sb)
            _wait(ones_v, cb)
            return carry

        lax.fori_loop(0, n_pairs, pair, 0)
        plsc.subcore_barrier()
        pltpu.sync_copy(ssum.at[pl.ds(row0, rpt)],
                        psum_hbm.at[cid, pl.ds(row0, rpt)])
        pltpu.sync_copy(scnt.at[pl.ds(row0, rpt)],
                        pcnt_hbm.at[cid, pl.ds(row0, rpt)])

    return edge


def _combine_kernel(n_acc, n_out):
    base = (n_out // _NW) // 8 * 8
    tail0 = _NW * base
    tail = n_out - tail0
    mesh = plsc.VectorSubcoreMesh(core_axis_name="c", subcore_axis_name="s")

    @functools.partial(
        pl.kernel,
        out_type=jax.ShapeDtypeStruct((n_out, D), jnp.float32),
        mesh=mesh,
        scratch_types=[
            pltpu.VMEM((base + tail, D), jnp.float32),
            pltpu.VMEM((base + tail, D), jnp.float32),
            pltpu.VMEM((base + tail, _CW), jnp.float32),
            pltpu.VMEM((base + tail, _CW), jnp.float32),
            pltpu.VMEM((base + tail, D), jnp.float32),
        ],
        compiler_params=pltpu.CompilerParams(use_tc_tiling_on_sc=False),
    )
    def combine(psum_hbm, pcnt_hbm, out_hbm, p0, p1, c0, c1, o):
        cid = lax.axis_index("c")
        sid = lax.axis_index("s")
        wid = cid * _NSUB + sid
        nr = jnp.where(wid == 0, base + tail, base)
        row0 = jnp.where(wid == 0, 0, wid * base + tail)

        def do(nrows):
            pltpu.sync_copy(psum_hbm.at[0, pl.ds(row0, nrows)],
                            p0.at[pl.ds(0, nrows)])
            pltpu.sync_copy(psum_hbm.at[1, pl.ds(row0, nrows)],
                            p1.at[pl.ds(0, nrows)])
            pltpu.sync_copy(pcnt_hbm.at[0, pl.ds(row0, nrows)],
                            c0.at[pl.ds(0, nrows)])
            pltpu.sync_copy(pcnt_hbm.at[1, pl.ds(row0, nrows)],
                            c1.at[pl.ds(0, nrows)])

            def one_row(r):
                inv = 1.0 / jnp.maximum(c0[r, pl.ds(0, 16)]
                                        + c1[r, pl.ds(0, 16)], 1.0)
                o[r, pl.ds(0, 16)] = (p0[r, pl.ds(0, 16)]
                                      + p1[r, pl.ds(0, 16)]) * inv
                o[r, pl.ds(16, 16)] = (p0[r, pl.ds(16, 16)]
                                       + p1[r, pl.ds(16, 16)]) * inv

            def body(r4, carry):
                for u in range(4):
                    one_row(r4 * 4 + u)
                return carry

            lax.fori_loop(0, nrows // 4, body, 0)
            pltpu.sync_copy(o.at[pl.ds(0, nrows)],
                            out_hbm.at[pl.ds(row0, nrows)])

        @pl.when(wid == 0)
        def _():
            do(base + tail)

        @pl.when(wid != 0)
        def _():
            do(base)

    return combine


def kernel(h, edge_index, Wq, Wk):
    n, d = h.shape
    e = edge_index.shape[1]

    a = _node_attention(h, Wq, Wk)

    n_acc = ((n + 1 + 127) // 128) * 128
    epw = ((e + _NW * _CH - 1) // (_NW * _CH)) * _CH
    n_chunks = epw // _CH
    e_pad = _NW * epw
    ei = jnp.pad(edge_index, ((0, 0), (0, e_pad - e)), constant_values=n)

    zsum = jnp.zeros((n_acc, D), jnp.float32)
    zcnt = jnp.zeros((n_acc, _CW), jnp.float32)
    ones = jnp.ones((_CH, _CW), jnp.float32)

    psum, pcnt = _edge_kernel(n_acc, n_chunks)(a, ei, zsum, zcnt, ones)
    return _combine_kernel(n_acc, n)(psum, pcnt)


# speedup vs baseline: 19.5151x; 1.0990x over previous
import functools

import jax
import jax.numpy as jnp
from jax import lax
from jax.experimental import pallas as pl
from jax.experimental.pallas import tpu as pltpu
from jax.experimental.pallas import tpu_sc as plsc

D = 32
_INV_SQRT_D = 1.0 / (32.0 ** 0.5)


_T = 1024


def _attn_body(h_ref, wq_ref, wk_ref, a_ref):
    hb = h_ref[...]
    hT = hb.T
    qT = jnp.dot(wq_ref[...], hT, preferred_element_type=jnp.float32)
    kT = jnp.dot(wk_ref[...], hT,
                 preferred_element_type=jnp.float32) * _INV_SQRT_D
    e3 = jnp.exp(qT[:, None, :] * kT[None, :, :])
    denom = jnp.sum(e3, axis=0)
    gT = hT / denom
    aT = jnp.sum(e3 * gT[None, :, :], axis=1)
    a_ref[...] = aT.T


def _node_attention(h_pad, wq, wk):
    grid = (h_pad.shape[0] + _T - 1) // _T
    n_pad = grid * _T
    return pl.pallas_call(
        _attn_body,
        grid=(grid,),
        in_specs=[
            pl.BlockSpec((_T, D), lambda i: (i, 0)),
            pl.BlockSpec((D, D), lambda i: (0, 0)),
            pl.BlockSpec((D, D), lambda i: (0, 0)),
        ],
        out_specs=pl.BlockSpec((_T, D), lambda i: (i, 0)),
        out_shape=jax.ShapeDtypeStruct((n_pad, D), jnp.float32),
        compiler_params=pltpu.CompilerParams(
            dimension_semantics=("arbitrary",)),
    )(h_pad, wq, wk)


_CH = 128
_NW = 32
_NSUB = 16
_CW = 16


_NBUF = 4


def _edge_kernel(n_acc, n_chunks):
    rpt = n_acc // _NSUB
    epw = n_chunks * _CH
    n_grp = n_chunks // _NBUF
    mesh = plsc.VectorSubcoreMesh(core_axis_name="c", subcore_axis_name="s")

    @functools.partial(
        pl.kernel,
        out_type=[
            jax.ShapeDtypeStruct((2, n_acc, D), jnp.float32),
            jax.ShapeDtypeStruct((2, n_acc, _CW), jnp.float32),
        ],
        mesh=mesh,
        scratch_types=[
            pltpu.VMEM((epw,), jnp.int32),
            pltpu.VMEM((epw,), jnp.int32),
            [pltpu.VMEM((_CH, D), jnp.float32)] * _NBUF,
            pltpu.VMEM((_CH, _CW), jnp.float32),
            pltpu.VMEM_SHARED((n_acc, D), jnp.float32),
            pltpu.VMEM_SHARED((n_acc, _CW), jnp.float32),
            [pltpu.SemaphoreType.DMA] * _NBUF,
            [pltpu.SemaphoreType.DMA] * _NBUF,
            [pltpu.SemaphoreType.DMA] * _NBUF,
        ],
        compiler_params=pltpu.CompilerParams(use_tc_tiling_on_sc=False),
    )
    def edge(a_hbm, ei_hbm, zsum_hbm, zcnt_hbm, ones_hbm,
             psum_hbm, pcnt_hbm,
             src_v, dst_v, rows, ones_v, ssum, scnt, gsem, ssem, csem):
        cid = lax.axis_index("c")
        sid = lax.axis_index("s")
        wid = cid * _NSUB + sid
        row0 = sid * rpt
        pltpu.sync_copy(zsum_hbm.at[pl.ds(row0, rpt)], ssum.at[pl.ds(row0, rpt)])
        pltpu.sync_copy(zcnt_hbm.at[pl.ds(row0, rpt)], scnt.at[pl.ds(row0, rpt)])
        pltpu.sync_copy(ones_hbm, ones_v)
        pltpu.sync_copy(ei_hbm.at[0, pl.ds(wid * epw, epw)], src_v)
        pltpu.sync_copy(ei_hbm.at[1, pl.ds(wid * epw, epw)], dst_v)
        plsc.subcore_barrier()

        def _gather(c, b):
            pltpu.async_copy(a_hbm.at[src_v.at[pl.ds(c * _CH, _CH)]],
                             rows[b], gsem[b])

        def _scatter(c, b):
            dix = dst_v.at[pl.ds(c * _CH, _CH)]
            pltpu.async_copy(rows[b], ssum.at[dix], ssem[b], add=True)
            pltpu.async_copy(ones_v, scnt.at[dix], csem[b], add=True)

        def _wait(buf, sem):
            if buf is ones_v:
                pltpu.make_async_copy(ones_hbm, buf, sem).wait()
            else:
                pltpu.make_async_copy(a_hbm.at[src_v.at[pl.ds(0, _CH)]],
                                      buf, sem).wait()

        for b in range(_NBUF):
            _gather(b, b)

        def group(i, carry):
            c0 = i * _NBUF
            for b in range(_NBUF):
                _wait(rows[b], gsem[b])
                _scatter(c0 + b, b)
            for b in range(_NBUF):
                _wait(rows[b], ssem[b])
                _wait(ones_v, csem[b])

                @pl.when(i + 1 < n_grp)
                def _():
                    _gather(c0 + _NBUF + b, b)
            return carry

        lax.fori_loop(0, n_grp, group, 0)
        plsc.subcore_barrier()
        pltpu.sync_copy(ssum.at[pl.ds(row0, rpt)],
                        psum_hbm.at[cid, pl.ds(row0, rpt)])
        pltpu.sync_copy(scnt.at[pl.ds(row0, rpt)],
                        pcnt_hbm.at[cid, pl.ds(row0, rpt)])

    return edge


def _combine_kernel(n_acc, n_out):
    base = (n_out // _NW) // 8 * 8
    tail0 = _NW * base
    tail = n_out - tail0
    mesh = plsc.VectorSubcoreMesh(core_axis_name="c", subcore_axis_name="s")

    @functools.partial(
        pl.kernel,
        out_type=jax.ShapeDtypeStruct((n_out, D), jnp.float32),
        mesh=mesh,
        scratch_types=[
            pltpu.VMEM((base + tail, D), jnp.float32),
            pltpu.VMEM((base + tail, D), jnp.float32),
            pltpu.VMEM((base + tail, _CW), jnp.float32),
            pltpu.VMEM((base + tail, _CW), jnp.float32),
            pltpu.VMEM((base + tail, D), jnp.float32),
            pltpu.SemaphoreType.DMA,
            pltpu.SemaphoreType.DMA,
        ],
        compiler_params=pltpu.CompilerParams(use_tc_tiling_on_sc=False),
    )
    def combine(psum_hbm, pcnt_hbm, out_hbm, p0, p1, c0, c1, o, psem, csem):
        cid = lax.axis_index("c")
        sid = lax.axis_index("s")
        wid = cid * _NSUB + sid
        nr = jnp.where(wid == 0, base + tail, base)
        row0 = jnp.where(wid == 0, 0, wid * base + tail)

        def do(nrows):
            cp0 = pltpu.async_copy(psum_hbm.at[0, pl.ds(row0, nrows)],
                                   p0.at[pl.ds(0, nrows)], psem)
            cp1 = pltpu.async_copy(psum_hbm.at[1, pl.ds(row0, nrows)],
                                   p1.at[pl.ds(0, nrows)], psem)
            cc0 = pltpu.async_copy(pcnt_hbm.at[0, pl.ds(row0, nrows)],
                                   c0.at[pl.ds(0, nrows)], csem)
            cc1 = pltpu.async_copy(pcnt_hbm.at[1, pl.ds(row0, nrows)],
                                   c1.at[pl.ds(0, nrows)], csem)
            cp0.wait()
            cp1.wait()
            cc0.wait()
            cc1.wait()

            def one_row(r):
                inv = 1.0 / jnp.maximum(c0[r, pl.ds(0, 16)]
                                        + c1[r, pl.ds(0, 16)], 1.0)
                o[r, pl.ds(0, 16)] = (p0[r, pl.ds(0, 16)]
                                      + p1[r, pl.ds(0, 16)]) * inv
                o[r, pl.ds(16, 16)] = (p0[r, pl.ds(16, 16)]
                                       + p1[r, pl.ds(16, 16)]) * inv

            def body(r4, carry):
                for u in range(4):
                    one_row(r4 * 4 + u)
                return carry

            lax.fori_loop(0, nrows // 4, body, 0)
            pltpu.sync_copy(o.at[pl.ds(0, nrows)],
                            out_hbm.at[pl.ds(row0, nrows)])

        @pl.when(wid == 0)
        def _():
            do(base + tail)

        @pl.when(wid != 0)
        def _():
            do(base)

    return combine


def kernel(h, edge_index, Wq, Wk):
    n, d = h.shape
    e = edge_index.shape[1]

    a = _node_attention(h, Wq, Wk)

    n_acc = ((n + 1 + 127) // 128) * 128
    grp = _NBUF * _CH
    epw = ((e + _NW * grp - 1) // (_NW * grp)) * grp
    n_chunks = epw // _CH
    e_pad = _NW * epw
    pads = jnp.arange(e_pad - e, dtype=jnp.int32)
    ei = jnp.concatenate(
        [edge_index, jnp.stack([pads % n, n + pads % (n_acc - n)])], axis=1)

    zsum = jnp.zeros((n_acc, D), jnp.float32)
    zcnt = jnp.zeros((n_acc, _CW), jnp.float32)
    ones = jnp.ones((_CH, _CW), jnp.float32)

    psum, pcnt = _edge_kernel(n_acc, n_chunks)(a, ei, zsum, zcnt, ones)
    return _combine_kernel(n_acc, n)(psum, pcnt)


# speedup vs baseline: 19.6755x; 1.0082x over previous
import functools

import jax
import jax.numpy as jnp
from jax import lax
from jax.experimental import pallas as pl
from jax.experimental.pallas import tpu as pltpu
from jax.experimental.pallas import tpu_sc as plsc

D = 32
_INV_SQRT_D = 1.0 / (32.0 ** 0.5)


_T = 1024


def _attn_body(h_ref, wq_ref, wk_ref, a_ref):
    hb = h_ref[...]
    hT = hb.T
    qT = jnp.dot(wq_ref[...], hT, preferred_element_type=jnp.float32)
    kT = jnp.dot(wk_ref[...], hT,
                 preferred_element_type=jnp.float32) * _INV_SQRT_D
    e3 = jnp.exp(qT[:, None, :] * kT[None, :, :])
    denom = jnp.sum(e3, axis=0)
    gT = hT / denom
    aT = jnp.sum(e3 * gT[None, :, :], axis=1)
    a_ref[...] = aT.T


def _node_attention(h_pad, wq, wk):
    grid = (h_pad.shape[0] + _T - 1) // _T
    n_pad = grid * _T
    return pl.pallas_call(
        _attn_body,
        grid=(grid,),
        in_specs=[
            pl.BlockSpec((_T, D), lambda i: (i, 0)),
            pl.BlockSpec((D, D), lambda i: (0, 0)),
            pl.BlockSpec((D, D), lambda i: (0, 0)),
        ],
        out_specs=pl.BlockSpec((_T, D), lambda i: (i, 0)),
        out_shape=jax.ShapeDtypeStruct((n_pad, D), jnp.float32),
        compiler_params=pltpu.CompilerParams(
            dimension_semantics=("arbitrary",)),
    )(h_pad, wq, wk)


_CH = 128
_NW = 32
_NSUB = 16
_CW = 16


_NBUF = 4


def _edge_kernel(n_acc, n_chunks):
    rpt = n_acc // _NSUB
    epw = n_chunks * _CH
    n_grp = n_chunks // _NBUF
    mesh = plsc.VectorSubcoreMesh(core_axis_name="c", subcore_axis_name="s")

    @functools.partial(
        pl.kernel,
        out_type=[
            jax.ShapeDtypeStruct((2, n_acc, D), jnp.float32),
            jax.ShapeDtypeStruct((2, n_acc, _CW), jnp.float32),
        ],
        mesh=mesh,
        scratch_types=[
            pltpu.VMEM((epw,), jnp.int32),
            pltpu.VMEM((epw,), jnp.int32),
            [pltpu.VMEM((_CH, D), jnp.float32)] * _NBUF,
            pltpu.VMEM((_CH, _CW), jnp.float32),
            pltpu.VMEM_SHARED((n_acc, D), jnp.float32),
            pltpu.VMEM_SHARED((n_acc, _CW), jnp.float32),
            [pltpu.SemaphoreType.DMA] * _NBUF,
            [pltpu.SemaphoreType.DMA] * _NBUF,
            [pltpu.SemaphoreType.DMA] * _NBUF,
        ],
        compiler_params=pltpu.CompilerParams(use_tc_tiling_on_sc=False,
                                            skip_device_barrier=True),
    )
    def edge(a_hbm, ei_hbm, zsum_hbm, zcnt_hbm, ones_hbm,
             psum_hbm, pcnt_hbm,
             src_v, dst_v, rows, ones_v, ssum, scnt, gsem, ssem, csem):
        cid = lax.axis_index("c")
        sid = lax.axis_index("s")
        wid = cid * _NSUB + sid
        row0 = sid * rpt
        pltpu.sync_copy(zsum_hbm.at[pl.ds(row0, rpt)], ssum.at[pl.ds(row0, rpt)])
        pltpu.sync_copy(zcnt_hbm.at[pl.ds(row0, rpt)], scnt.at[pl.ds(row0, rpt)])
        pltpu.sync_copy(ones_hbm, ones_v)
        pltpu.sync_copy(ei_hbm.at[0, pl.ds(wid * epw, epw)], src_v)
        pltpu.sync_copy(ei_hbm.at[1, pl.ds(wid * epw, epw)], dst_v)
        plsc.subcore_barrier()

        def _gather(c, b):
            pltpu.async_copy(a_hbm.at[src_v.at[pl.ds(c * _CH, _CH)]],
                             rows[b], gsem[b])

        def _scatter(c, b):
            dix = dst_v.at[pl.ds(c * _CH, _CH)]
            pltpu.async_copy(rows[b], ssum.at[dix], ssem[b], add=True)
            pltpu.async_copy(ones_v, scnt.at[dix], csem[b], add=True)

        def _wait(buf, sem):
            if buf is ones_v:
                pltpu.make_async_copy(ones_hbm, buf, sem).wait()
            else:
                pltpu.make_async_copy(a_hbm.at[src_v.at[pl.ds(0, _CH)]],
                                      buf, sem).wait()

        for b in range(_NBUF):
            _gather(b, b)

        def group(i, carry):
            c0 = i * _NBUF
            for b in range(_NBUF):
                _wait(rows[b], gsem[b])
                _scatter(c0 + b, b)
            for b in range(_NBUF):
                _wait(rows[b], ssem[b])
                _wait(ones_v, csem[b])

                @pl.when(i + 1 < n_grp)
                def _():
                    _gather(c0 + _NBUF + b, b)
            return carry

        lax.fori_loop(0, n_grp, group, 0)
        c0 = n_grp * _NBUF
        n_rem = n_chunks - c0
        for b in range(n_rem):
            _gather(c0 + b, b)
        for b in range(n_rem):
            _wait(rows[b], gsem[b])
            _scatter(c0 + b, b)
        for b in range(n_rem):
            _wait(rows[b], ssem[b])
            _wait(ones_v, csem[b])
        plsc.subcore_barrier()
        pltpu.sync_copy(ssum.at[pl.ds(row0, rpt)],
                        psum_hbm.at[cid, pl.ds(row0, rpt)])
        pltpu.sync_copy(scnt.at[pl.ds(row0, rpt)],
                        pcnt_hbm.at[cid, pl.ds(row0, rpt)])

    return edge


def _combine_kernel(n_acc, n_out):
    base = (n_out // _NW) // 8 * 8
    tail0 = _NW * base
    tail = n_out - tail0
    mesh = plsc.VectorSubcoreMesh(core_axis_name="c", subcore_axis_name="s")

    @functools.partial(
        pl.kernel,
        out_type=jax.ShapeDtypeStruct((n_out, D), jnp.float32),
        mesh=mesh,
        scratch_types=[
            pltpu.VMEM((base + tail, D), jnp.float32),
            pltpu.VMEM((base + tail, D), jnp.float32),
            pltpu.VMEM((base + tail, _CW), jnp.float32),
            pltpu.VMEM((base + tail, _CW), jnp.float32),
            pltpu.VMEM((base + tail, D), jnp.float32),
            pltpu.SemaphoreType.DMA,
            pltpu.SemaphoreType.DMA,
        ],
        compiler_params=pltpu.CompilerParams(use_tc_tiling_on_sc=False,
                                            skip_device_barrier=True),
    )
    def combine(psum_hbm, pcnt_hbm, out_hbm, p0, p1, c0, c1, o, psem, csem):
        cid = lax.axis_index("c")
        sid = lax.axis_index("s")
        wid = cid * _NSUB + sid
        nr = jnp.where(wid == 0, base + tail, base)
        row0 = jnp.where(wid == 0, 0, wid * base + tail)

        def do(nrows):
            cp0 = pltpu.async_copy(psum_hbm.at[0, pl.ds(row0, nrows)],
                                   p0.at[pl.ds(0, nrows)], psem)
            cp1 = pltpu.async_copy(psum_hbm.at[1, pl.ds(row0, nrows)],
                                   p1.at[pl.ds(0, nrows)], psem)
            cc0 = pltpu.async_copy(pcnt_hbm.at[0, pl.ds(row0, nrows)],
                                   c0.at[pl.ds(0, nrows)], csem)
            cc1 = pltpu.async_copy(pcnt_hbm.at[1, pl.ds(row0, nrows)],
                                   c1.at[pl.ds(0, nrows)], csem)
            cp0.wait()
            cp1.wait()
            cc0.wait()
            cc1.wait()

            def one_row(r):
                inv = 1.0 / jnp.maximum(c0[r, pl.ds(0, 16)]
                                        + c1[r, pl.ds(0, 16)], 1.0)
                o[r, pl.ds(0, 16)] = (p0[r, pl.ds(0, 16)]
                                      + p1[r, pl.ds(0, 16)]) * inv
                o[r, pl.ds(16, 16)] = (p0[r, pl.ds(16, 16)]
                                       + p1[r, pl.ds(16, 16)]) * inv

            def body(r4, carry):
                for u in range(4):
                    one_row(r4 * 4 + u)
                return carry

            lax.fori_loop(0, nrows // 4, body, 0)
            pltpu.sync_copy(o.at[pl.ds(0, nrows)],
                            out_hbm.at[pl.ds(row0, nrows)])

        @pl.when(wid == 0)
        def _():
            do(base + tail)

        @pl.when(wid != 0)
        def _():
            do(base)

    return combine


def kernel(h, edge_index, Wq, Wk):
    n, d = h.shape
    e = edge_index.shape[1]

    a = _node_attention(h, Wq, Wk)

    n_acc = ((n + 1 + 127) // 128) * 128
    epw = ((e + _NW * _CH - 1) // (_NW * _CH)) * _CH
    n_chunks = epw // _CH
    e_pad = _NW * epw
    ei = jnp.pad(edge_index, ((0, 0), (0, e_pad - e)), constant_values=n)

    zsum = jnp.zeros((n_acc, D), jnp.float32)
    zcnt = jnp.zeros((n_acc, _CW), jnp.float32)
    ones = jnp.ones((_CH, _CW), jnp.float32)

    psum, pcnt = _edge_kernel(n_acc, n_chunks)(a, ei, zsum, zcnt, ones)
    return _combine_kernel(n_acc, n)(psum, pcnt)
